# baseline (speedup 1.0000x reference)
import functools

import jax
import jax.numpy as jnp
from jax import lax
from jax.experimental import pallas as pl
from jax.experimental.pallas import tpu as pltpu
from jax.experimental.pallas import tpu_sc as plsc

NUM_TYPES = 4
VOCAB = 100000
EMBED = 32
N = 425984

NC = 2
NS = 16
NW = NC * NS
B_PER_W = N // NW
CHUNK = 256
N_CHUNKS = B_PER_W // CHUNK

_mesh = plsc.VectorSubcoreMesh(core_axis_name="c", subcore_axis_name="s")


@functools.partial(
    pl.kernel,
    mesh=_mesh,
    out_type=jax.ShapeDtypeStruct((EMBED, N), jnp.float32),
    compiler_params=pltpu.CompilerParams(use_tc_tiling_on_sc=True,
                                         needs_layout_passes=False),
    scratch_types=[
        [pltpu.VMEM((CHUNK,), jnp.int32)] * 2,
        [pltpu.VMEM((CHUNK,), jnp.int32)] * 2,
        [pltpu.VMEM((CHUNK, 4 * EMBED), jnp.float32)] * 2,
        [pltpu.VMEM((EMBED, CHUNK), jnp.float32)] * 2,
        [pltpu.SemaphoreType.DMA] * 2,
        [pltpu.SemaphoreType.DMA] * 2,
        [pltpu.SemaphoreType.DMA] * 2,
    ],
)
def _hetero_gather(x_hbm, types_hbm, table_hbm, out_hbm,
                   x_v, t_v, rows_v, seg_v, sem_xt, sem_g, sem_o):
    wid = lax.axis_index("s") * NC + lax.axis_index("c")
    base_w = wid * B_PER_W
    iota = lax.iota(jnp.int32, 16)

    def fire_xt(c, b):
        base = base_w + c * CHUNK
        pltpu.async_copy(x_hbm.at[pl.ds(base, CHUNK)], x_v[b], sem_xt[b])
        pltpu.async_copy(types_hbm.at[pl.ds(base, CHUNK)], t_v[b], sem_xt[b])

    def wait_xt(c, b):
        base = base_w + c * CHUNK
        pltpu.make_async_copy(x_hbm.at[pl.ds(base, CHUNK)], x_v[b], sem_xt[b]).wait()
        pltpu.make_async_copy(types_hbm.at[pl.ds(base, CHUNK)], t_v[b], sem_xt[b]).wait()

    def out_block(c):
        base = base_w + c * CHUNK
        return out_hbm.at[:, pl.ds(base, CHUNK)]

    fire_xt(0, 0)
    fire_xt(1, 1)
    wait_xt(0, 0)
    pltpu.async_copy(table_hbm.at[x_v[0]], rows_v[0], sem_g[0])

    def outer(g, carry):
        for b in range(2):
            c = g * 2 + b
            nb = 1 - b

            @pl.when(c + 1 < N_CHUNKS)
            def _():
                wait_xt(c + 1, nb)
                pltpu.async_copy(table_hbm.at[x_v[nb]], rows_v[nb], sem_g[nb])

            pltpu.make_async_copy(table_hbm.at[x_v[b]], rows_v[b], sem_g[b]).wait()

            @pl.when(c >= 2)
            def _():
                pltpu.make_async_copy(seg_v[b], out_block(c - 2), sem_o[b]).wait()

            def group_body(i, carry2):
                r0 = i * 16
                rowv = r0 + iota
                colbase = t_v[b][pl.ds(r0, 16)] * EMBED
                for d in range(EMBED):
                    g16 = plsc.load_gather(rows_v[b], [rowv, colbase + d])
                    seg_v[b][d, pl.ds(r0, 16)] = g16
                return carry2

            lax.fori_loop(0, CHUNK // 16, group_body, 0)

            @pl.when(c + 2 < N_CHUNKS)
            def _():
                fire_xt(c + 2, b)

            pltpu.async_copy(seg_v[b], out_block(c), sem_o[b])
        return carry

    lax.fori_loop(0, N_CHUNKS // 2, outer, 0)

    for b in range(2):
        pltpu.make_async_copy(seg_v[b], out_block(N_CHUNKS - 2 + b), sem_o[b]).wait()


def kernel(x, types, table_0, table_1, table_2, table_3):
    table = jnp.concatenate([table_0, table_1, table_2, table_3], axis=1)
    out_t = _hetero_gather(x.astype(jnp.int32), types.astype(jnp.int32), table)
    return out_t.T


# speedup vs baseline: 1.5513x; 1.5513x over previous
import functools

import jax
import jax.numpy as jnp
from jax import lax
from jax.experimental import pallas as pl
from jax.experimental.pallas import tpu as pltpu
from jax.experimental.pallas import tpu_sc as plsc

NUM_TYPES = 4
VOCAB = 100000
EMBED = 32
N = 425984

NC = 2
NS = 16
NW = NC * NS
B_PER_W = N // NW
CHUNK = 256
N_CHUNKS = B_PER_W // CHUNK

_mesh = plsc.VectorSubcoreMesh(core_axis_name="c", subcore_axis_name="s")


@functools.partial(
    pl.kernel,
    mesh=_mesh,
    out_type=jax.ShapeDtypeStruct((EMBED, N), jnp.float32),
    compiler_params=pltpu.CompilerParams(use_tc_tiling_on_sc=True,
                                         needs_layout_passes=False),
    scratch_types=[
        [pltpu.VMEM((CHUNK,), jnp.int32)] * 2,
        [pltpu.VMEM((CHUNK,), jnp.int32)] * 2,
        [pltpu.VMEM((CHUNK, 4 * EMBED), jnp.float32)] * 2,
        [pltpu.VMEM((EMBED, CHUNK), jnp.float32)] * 2,
        [pltpu.SemaphoreType.DMA] * 2,
        [pltpu.SemaphoreType.DMA] * 2,
        [pltpu.SemaphoreType.DMA] * 2,
    ],
)
def _hetero_gather(x_hbm, types_hbm, table_hbm, out_hbm,
                   x_v, t_v, rows_v, seg_v, sem_xt, sem_g, sem_o):
    wid = lax.axis_index("s") * NC + lax.axis_index("c")
    base_w = wid * B_PER_W
    iota = lax.iota(jnp.int32, 16)

    def fire_xt(c, b):
        base = base_w + c * CHUNK
        pltpu.async_copy(x_hbm.at[pl.ds(base, CHUNK)], x_v[b], sem_xt[b])
        pltpu.async_copy(types_hbm.at[pl.ds(base, CHUNK)], t_v[b], sem_xt[b])

    def wait_xt(c, b):
        base = base_w + c * CHUNK
        pltpu.make_async_copy(x_hbm.at[pl.ds(base, CHUNK)], x_v[b], sem_xt[b]).wait()
        pltpu.make_async_copy(types_hbm.at[pl.ds(base, CHUNK)], t_v[b], sem_xt[b]).wait()

    def out_block(c):
        base = base_w + c * CHUNK
        return out_hbm.at[:, pl.ds(base, CHUNK)]

    fire_xt(0, 0)
    fire_xt(1, 1)
    wait_xt(0, 0)
    pltpu.async_copy(table_hbm.at[x_v[0]], rows_v[0], sem_g[0])

    def outer(g, carry):
        for b in range(2):
            c = g * 2 + b
            nb = 1 - b

            @pl.when(c + 1 < N_CHUNKS)
            def _():
                wait_xt(c + 1, nb)
                pltpu.async_copy(table_hbm.at[x_v[nb]], rows_v[nb], sem_g[nb])

            pltpu.make_async_copy(table_hbm.at[x_v[b]], rows_v[b], sem_g[b]).wait()

            @pl.when(c >= 2)
            def _():
                pltpu.make_async_copy(seg_v[b], out_block(c - 2), sem_o[b]).wait()

            def group_body(i, carry2):
                r0 = i * 16
                rowv = r0 + iota
                colbase = t_v[b][pl.ds(r0, 16)] * EMBED
                for c0 in range(0, EMBED, 16):
                    tbase = colbase + c0
                    for k in range(16):
                        dk = (iota + k) & 15
                        g16 = plsc.load_gather(rows_v[b], [rowv, tbase + dk])
                        plsc.store_scatter(seg_v[b], [c0 + dk, rowv], g16)
                return carry2

            lax.fori_loop(0, CHUNK // 16, group_body, 0)

            @pl.when(c + 2 < N_CHUNKS)
            def _():
                fire_xt(c + 2, b)

            pltpu.async_copy(seg_v[b], out_block(c), sem_o[b])
        return carry

    lax.fori_loop(0, N_CHUNKS // 2, outer, 0)

    for b in range(2):
        pltpu.make_async_copy(seg_v[b], out_block(N_CHUNKS - 2 + b), sem_o[b]).wait()


def kernel(x, types, table_0, table_1, table_2, table_3):
    table = jnp.concatenate([table_0, table_1, table_2, table_3], axis=1)
    out_t = _hetero_gather(x.astype(jnp.int32), types.astype(jnp.int32), table)
    return out_t.T


# speedup vs baseline: 1.6983x; 1.0948x over previous
import functools

import jax
import jax.numpy as jnp
from jax import lax
from jax.experimental import pallas as pl
from jax.experimental.pallas import tpu as pltpu
from jax.experimental.pallas import tpu_sc as plsc

NUM_TYPES = 4
VOCAB = 100000
EMBED = 32
N = 425984

NC = 2
NS = 16
NW = NC * NS
B_PER_W = N // NW
CHUNK = 256
N_CHUNKS = B_PER_W // CHUNK

VPAD = 100096
VTILES = VPAD // 128

_mesh = plsc.VectorSubcoreMesh(core_axis_name="c", subcore_axis_name="s")


@functools.partial(
    pl.kernel,
    mesh=_mesh,
    out_type=jax.ShapeDtypeStruct((VPAD, 4 * EMBED), jnp.float32),
    compiler_params=pltpu.CompilerParams(use_tc_tiling_on_sc=True,
                                         needs_layout_passes=False),
    scratch_types=[
        [pltpu.VMEM((4 * EMBED, 128), jnp.float32)] * 2,
        [pltpu.VMEM((128, 4 * EMBED), jnp.float32)] * 2,
        [pltpu.SemaphoreType.DMA] * 2,
        [pltpu.SemaphoreType.DMA] * 2,
    ],
)
def _transpose_concat(t0_hbm, t1_hbm, t2_hbm, t3_hbm, big_hbm,
                      src_v, dst_v, sem_s, sem_d):
    wid = lax.axis_index("s") * NC + lax.axis_index("c")
    iota = lax.iota(jnp.int32, 16)
    tabs = (t0_hbm, t1_hbm, t2_hbm, t3_hbm)
    jobs = VTILES // NW + 1

    def fire_src(j, b):
        v0 = j * 128
        for t in range(4):
            for p in range(4):
                pltpu.async_copy(tabs[t].at[pl.ds(8 * p, 8), pl.ds(v0, 128)],
                                 src_v[b].at[pl.ds(t * EMBED + 8 * p, 8), :],
                                 sem_s[b])

    def wait_src(j, b):
        v0 = j * 128
        for t in range(4):
            for p in range(4):
                pltpu.make_async_copy(
                    tabs[t].at[pl.ds(8 * p, 8), pl.ds(v0, 128)],
                    src_v[b].at[pl.ds(t * EMBED + 8 * p, 8), :],
                    sem_s[b]).wait()

    @pl.when(wid < VTILES)
    def _():
        fire_src(wid, 0)

    @pl.when(wid + NW < VTILES)
    def _():
        fire_src(wid + NW, 1)

    def body(i, carry):
        for b in range(2):
            jl = i * 2 + b
            j = wid + jl * NW

            @pl.when(j < VTILES)
            def _():
                wait_src(j, b)

                @pl.when(j >= 2 * NW)
                def _():
                    pltpu.make_async_copy(
                        dst_v[b], big_hbm.at[pl.ds((j - 2 * NW) * 128, 128), :],
                        sem_d[b]).wait()

                def block(g, carry2):
                    d0 = (g % 8) * 16
                    v0 = (g // 8) * 16
                    for k in range(16):
                        dk = (iota + k) & 15
                        g16 = plsc.load_gather(src_v[b], [d0 + dk, v0 + iota])
                        plsc.store_scatter(dst_v[b], [v0 + iota, d0 + dk], g16)
                    return carry2

                lax.fori_loop(0, 64, block, 0)

                pltpu.async_copy(dst_v[b], big_hbm.at[pl.ds(j * 128, 128), :],
                                 sem_d[b])

            @pl.when(j + 2 * NW < VTILES)
            def _():
                fire_src(j + 2 * NW, b)
        return carry

    lax.fori_loop(0, (jobs + 1) // 2, body, 0)

    for b in range(2):
        jl_last = max(jl for jl in range(jobs) if jl % 2 == b)
        j_last = wid + jl_last * NW

        @pl.when(j_last < VTILES)
        def _():
            pltpu.make_async_copy(dst_v[b],
                                  big_hbm.at[pl.ds(j_last * 128, 128), :],
                                  sem_d[b]).wait()


@functools.partial(
    pl.kernel,
    mesh=_mesh,
    out_type=jax.ShapeDtypeStruct((EMBED, N), jnp.float32),
    compiler_params=pltpu.CompilerParams(use_tc_tiling_on_sc=True,
                                         needs_layout_passes=False),
    scratch_types=[
        [pltpu.VMEM((CHUNK,), jnp.int32)] * 2,
        [pltpu.VMEM((CHUNK,), jnp.int32)] * 2,
        [pltpu.VMEM((CHUNK, 4 * EMBED), jnp.float32)] * 2,
        [pltpu.VMEM((EMBED, CHUNK), jnp.float32)] * 2,
        [pltpu.SemaphoreType.DMA] * 2,
        [pltpu.SemaphoreType.DMA] * 2,
        [pltpu.SemaphoreType.DMA] * 2,
    ],
)
def _hetero_gather(x_hbm, types_hbm, table_hbm, out_hbm,
                   x_v, t_v, rows_v, seg_v, sem_xt, sem_g, sem_o):
    wid = lax.axis_index("s") * NC + lax.axis_index("c")
    base_w = wid * B_PER_W
    iota = lax.iota(jnp.int32, 16)

    def fire_xt(c, b):
        base = base_w + c * CHUNK
        pltpu.async_copy(x_hbm.at[pl.ds(base, CHUNK)], x_v[b], sem_xt[b])
        pltpu.async_copy(types_hbm.at[pl.ds(base, CHUNK)], t_v[b], sem_xt[b])

    def wait_xt(c, b):
        base = base_w + c * CHUNK
        pltpu.make_async_copy(x_hbm.at[pl.ds(base, CHUNK)], x_v[b], sem_xt[b]).wait()
        pltpu.make_async_copy(types_hbm.at[pl.ds(base, CHUNK)], t_v[b], sem_xt[b]).wait()

    def out_block(c):
        base = base_w + c * CHUNK
        return out_hbm.at[:, pl.ds(base, CHUNK)]

    fire_xt(0, 0)
    fire_xt(1, 1)
    wait_xt(0, 0)
    pltpu.async_copy(table_hbm.at[x_v[0]], rows_v[0], sem_g[0])

    def outer(g, carry):
        for b in range(2):
            c = g * 2 + b
            nb = 1 - b

            @pl.when(c + 1 < N_CHUNKS)
            def _():
                wait_xt(c + 1, nb)
                pltpu.async_copy(table_hbm.at[x_v[nb]], rows_v[nb], sem_g[nb])

            pltpu.make_async_copy(table_hbm.at[x_v[b]], rows_v[b], sem_g[b]).wait()

            @pl.when(c >= 2)
            def _():
                pltpu.make_async_copy(seg_v[b], out_block(c - 2), sem_o[b]).wait()

            def group_body(i, carry2):
                r0 = i * 16
                rowv = r0 + iota
                colbase = t_v[b][pl.ds(r0, 16)] * EMBED
                for c0 in range(0, EMBED, 16):
                    tbase = colbase + c0
                    for k in range(16):
                        dk = (iota + k) & 15
                        g16 = plsc.load_gather(rows_v[b], [rowv, tbase + dk])
                        plsc.store_scatter(seg_v[b], [c0 + dk, rowv], g16)
                return carry2

            lax.fori_loop(0, CHUNK // 16, group_body, 0)

            @pl.when(c + 2 < N_CHUNKS)
            def _():
                fire_xt(c + 2, b)

            pltpu.async_copy(seg_v[b], out_block(c), sem_o[b])
        return carry

    lax.fori_loop(0, N_CHUNKS // 2, outer, 0)

    for b in range(2):
        pltpu.make_async_copy(seg_v[b], out_block(N_CHUNKS - 2 + b), sem_o[b]).wait()


def kernel(x, types, table_0, table_1, table_2, table_3):
    table = _transpose_concat(table_0.T, table_1.T, table_2.T, table_3.T)
    out_t = _hetero_gather(x.astype(jnp.int32), types.astype(jnp.int32), table)
    return out_t.T


# speedup vs baseline: 1.7018x; 1.0020x over previous
import functools

import jax
import jax.numpy as jnp
from jax import lax
from jax.experimental import pallas as pl
from jax.experimental.pallas import tpu as pltpu
from jax.experimental.pallas import tpu_sc as plsc

NUM_TYPES = 4
VOCAB = 100000
EMBED = 32
N = 425984

NC = 2
NS = 16
NW = NC * NS
B_PER_W = N // NW
CHUNK = 256
N_CHUNKS = B_PER_W // CHUNK
NBUF = 2

VPAD = 100096
VTILES = VPAD // 128

_mesh = plsc.VectorSubcoreMesh(core_axis_name="c", subcore_axis_name="s")


@functools.partial(
    pl.kernel,
    mesh=_mesh,
    out_type=jax.ShapeDtypeStruct((VPAD, 4 * EMBED), jnp.float32),
    compiler_params=pltpu.CompilerParams(use_tc_tiling_on_sc=True,
                                         needs_layout_passes=False),
    scratch_types=[
        [pltpu.VMEM((4 * EMBED, 128), jnp.float32)] * 2,
        [pltpu.VMEM((128, 4 * EMBED), jnp.float32)] * 2,
        [pltpu.SemaphoreType.DMA] * 2,
        [pltpu.SemaphoreType.DMA] * 2,
    ],
)
def _transpose_concat(t0_hbm, t1_hbm, t2_hbm, t3_hbm, big_hbm,
                      src_v, dst_v, sem_s, sem_d):
    wid = lax.axis_index("s") * NC + lax.axis_index("c")
    iota = lax.iota(jnp.int32, 16)
    tabs = (t0_hbm, t1_hbm, t2_hbm, t3_hbm)
    jobs = VTILES // NW + 1

    def fire_src(j, b):
        v0 = j * 128
        for t in range(4):
            for p in range(4):
                pltpu.async_copy(tabs[t].at[pl.ds(8 * p, 8), pl.ds(v0, 128)],
                                 src_v[b].at[pl.ds(t * EMBED + 8 * p, 8), :],
                                 sem_s[b])

    def wait_src(j, b):
        v0 = j * 128
        for t in range(4):
            for p in range(4):
                pltpu.make_async_copy(
                    tabs[t].at[pl.ds(8 * p, 8), pl.ds(v0, 128)],
                    src_v[b].at[pl.ds(t * EMBED + 8 * p, 8), :],
                    sem_s[b]).wait()

    @pl.when(wid < VTILES)
    def _():
        fire_src(wid, 0)

    @pl.when(wid + NW < VTILES)
    def _():
        fire_src(wid + NW, 1)

    def body(i, carry):
        for b in range(2):
            jl = i * 2 + b
            j = wid + jl * NW

            @pl.when(j < VTILES)
            def _():
                wait_src(j, b)

                @pl.when(j >= 2 * NW)
                def _():
                    pltpu.make_async_copy(
                        dst_v[b], big_hbm.at[pl.ds((j - 2 * NW) * 128, 128), :],
                        sem_d[b]).wait()

                def block(g, carry2):
                    d0 = (g % 8) * 16
                    v0 = (g // 8) * 16
                    for k in range(16):
                        dk = (iota + k) & 15
                        g16 = plsc.load_gather(src_v[b], [d0 + dk, v0 + iota])
                        plsc.store_scatter(dst_v[b], [v0 + iota, d0 + dk], g16)
                    return carry2

                lax.fori_loop(0, 64, block, 0)

                pltpu.async_copy(dst_v[b], big_hbm.at[pl.ds(j * 128, 128), :],
                                 sem_d[b])

            @pl.when(j + 2 * NW < VTILES)
            def _():
                fire_src(j + 2 * NW, b)
        return carry

    lax.fori_loop(0, (jobs + 1) // 2, body, 0)

    for b in range(2):
        jl_last = max(jl for jl in range(jobs) if jl % 2 == b)
        j_last = wid + jl_last * NW

        @pl.when(j_last < VTILES)
        def _():
            pltpu.make_async_copy(dst_v[b],
                                  big_hbm.at[pl.ds(j_last * 128, 128), :],
                                  sem_d[b]).wait()


@functools.partial(
    pl.kernel,
    mesh=_mesh,
    out_type=jax.ShapeDtypeStruct((EMBED, N), jnp.float32),
    compiler_params=pltpu.CompilerParams(use_tc_tiling_on_sc=True,
                                         needs_layout_passes=False),
    scratch_types=[
        [pltpu.VMEM((CHUNK,), jnp.int32)] * NBUF,
        [pltpu.VMEM((CHUNK,), jnp.int32)] * NBUF,
        [pltpu.VMEM((CHUNK, 4 * EMBED), jnp.float32)] * NBUF,
        [pltpu.VMEM((EMBED, CHUNK), jnp.float32)] * NBUF,
        [pltpu.SemaphoreType.DMA] * NBUF,
        [pltpu.SemaphoreType.DMA] * NBUF,
        [pltpu.SemaphoreType.DMA] * NBUF,
    ],
)
def _hetero_gather(x_hbm, types_hbm, table_hbm, out_hbm,
                   x_v, t_v, rows_v, seg_v, sem_xt, sem_g, sem_o):
    wid = lax.axis_index("s") * NC + lax.axis_index("c")
    base_w = wid * B_PER_W
    iota = lax.iota(jnp.int32, 16)

    def fire_xt(c, b):
        base = base_w + c * CHUNK
        pltpu.async_copy(x_hbm.at[pl.ds(base, CHUNK)], x_v[b], sem_xt[b])
        pltpu.async_copy(types_hbm.at[pl.ds(base, CHUNK)], t_v[b], sem_xt[b])

    def wait_xt(c, b):
        base = base_w + c * CHUNK
        pltpu.make_async_copy(x_hbm.at[pl.ds(base, CHUNK)], x_v[b], sem_xt[b]).wait()
        pltpu.make_async_copy(types_hbm.at[pl.ds(base, CHUNK)], t_v[b], sem_xt[b]).wait()

    def out_block(c):
        base = base_w + c * CHUNK
        return out_hbm.at[:, pl.ds(base, CHUNK)]

    def fire_gather(b):
        pltpu.async_copy(table_hbm.at[x_v[b]], rows_v[b], sem_g[b])

    for b in range(NBUF):
        fire_xt(b, b)
    wait_xt(0, 0)
    fire_gather(0)

    def outer(g, carry):
        for b in range(NBUF):
            c = g * NBUF + b
            nb = 1 - b

            @pl.when(c + 1 < N_CHUNKS)
            def _():
                wait_xt(c + 1, nb)
                fire_gather(nb)

            pltpu.make_async_copy(table_hbm.at[x_v[b]], rows_v[b], sem_g[b]).wait()

            @pl.when(c >= NBUF)
            def _():
                pltpu.make_async_copy(seg_v[b], out_block(c - NBUF), sem_o[b]).wait()

            def group_body(i, carry2):
                r0 = i * 16
                rowv = r0 + iota
                colbase = t_v[b][pl.ds(r0, 16)] * EMBED
                for c0 in range(0, EMBED, 16):
                    tbase = colbase + c0
                    for k in range(16):
                        dk = (iota + k) & 15
                        g16 = plsc.load_gather(rows_v[b], [rowv, tbase + dk])
                        plsc.store_scatter(seg_v[b], [c0 + dk, rowv], g16)
                return carry2

            lax.fori_loop(0, CHUNK // 16, group_body, 0)

            @pl.when(c + NBUF < N_CHUNKS)
            def _():
                fire_xt(c + NBUF, b)

            pltpu.async_copy(seg_v[b], out_block(c), sem_o[b])
        return carry

    lax.fori_loop(0, N_CHUNKS // NBUF, outer, 0)

    for b in range(NBUF):
        pltpu.make_async_copy(seg_v[b], out_block(N_CHUNKS - NBUF + b), sem_o[b]).wait()


def kernel(x, types, table_0, table_1, table_2, table_3):
    table = _transpose_concat(table_0.T, table_1.T, table_2.T, table_3.T)
    out_t = _hetero_gather(x.astype(jnp.int32), types.astype(jnp.int32), table)
    return out_t.T
